# baseline (speedup 1.0000x reference)
import functools

import jax
import jax.numpy as jnp
from jax import lax
from jax.experimental import pallas as pl
from jax.experimental.pallas import tpu as pltpu
from jax.experimental.pallas import tpu_sc as plsc

_NC, _NS, _LANES = 2, 16, 16
_NW = _NC * _NS


def _make_sc_call(B, A, N, L):
    G = B * A
    CL = L // _NW
    assert L % _NW == 0 and CL % _LANES == 0

    mesh = plsc.VectorSubcoreMesh(
        core_axis_name="c", subcore_axis_name="s",
        num_cores=_NC, num_subcores=_NS)

    @functools.partial(
        pl.kernel,
        out_type=(
            jax.ShapeDtypeStruct((G, L), jnp.float32),
            jax.ShapeDtypeStruct((G, L), jnp.int32),
        ),
        mesh=mesh,
        scratch_types=[
            pltpu.VMEM((B, CL), jnp.float32),
            pltpu.VMEM((N, CL), jnp.float32),
            pltpu.VMEM((CL,), jnp.float32),
            pltpu.VMEM((CL,), jnp.int32),
        ],
    )
    def sc_call(mixed_hbm, ref_hbm, omax_hbm, oidx_hbm, m_v, r_v, omax_v, oidx_v):
        wid = lax.axis_index("s") * _NC + lax.axis_index("c")
        col0 = wid * CL
        pltpu.sync_copy(mixed_hbm.at[:, pl.ds(col0, CL)], m_v)
        for g in range(G):
            b = g // A
            pltpu.sync_copy(ref_hbm.at[pl.ds(g * N, N), pl.ds(col0, CL)], r_v)

            @pl.loop(0, CL // _LANES)
            def _(j):
                sl = pl.ds(j * _LANES, _LANES)
                mv = m_v[b, sl]
                best = mv * r_v[0, sl]
                besti = jnp.zeros((_LANES,), jnp.int32)
                for n in range(1, N):
                    p = mv * r_v[n, sl]
                    gt = p > best
                    best = jnp.where(gt, p, best)
                    besti = jnp.where(gt, jnp.int32(n), besti)
                omax_v[sl] = best
                oidx_v[sl] = besti

            pltpu.sync_copy(omax_v, omax_hbm.at[g, pl.ds(col0, CL)])
            pltpu.sync_copy(oidx_v, oidx_hbm.at[g, pl.ds(col0, CL)])

    return sc_call


def kernel(input_mixed, ref_panel):
    B, A, N, L = ref_panel.shape
    sc_call = _make_sc_call(B, A, N, L)
    ref2d = ref_panel.reshape(B * A * N, L)
    omax, oidx = sc_call(input_mixed, ref2d)
    return omax.reshape(B, A, L), oidx.reshape(B, A, L)


# speedup vs baseline: 1.3984x; 1.3984x over previous
import functools

import jax
import jax.numpy as jnp
from jax import lax
from jax.experimental import pallas as pl
from jax.experimental.pallas import tpu as pltpu
from jax.experimental.pallas import tpu_sc as plsc

_NC, _NS, _LANES = 2, 16, 16
_NW = _NC * _NS


def _make_sc_call(B, A, N, L):
    G = B * A
    CL = L // _NW
    assert L % _NW == 0 and CL % _LANES == 0

    mesh = plsc.VectorSubcoreMesh(
        core_axis_name="c", subcore_axis_name="s",
        num_cores=_NC, num_subcores=_NS)

    @functools.partial(
        pl.kernel,
        out_type=(
            jax.ShapeDtypeStruct((G, L), jnp.float32),
            jax.ShapeDtypeStruct((G, L), jnp.int32),
        ),
        mesh=mesh,
        scratch_types=[
            pltpu.VMEM((B, CL), jnp.float32),
            pltpu.VMEM((2, N, CL), jnp.float32),
            pltpu.VMEM((G, CL), jnp.float32),
            pltpu.VMEM((G, CL), jnp.int32),
            pltpu.SemaphoreType.DMA,
            pltpu.SemaphoreType.DMA,
        ],
    )
    def sc_call(mixed_hbm, ref_hbm, omax_hbm, oidx_hbm, m_v, r_v, omax_v,
                oidx_v, sr0, sr1):
        srs = (sr0, sr1)
        wid = lax.axis_index("s") * _NC + lax.axis_index("c")
        col0 = wid * CL
        pltpu.sync_copy(mixed_hbm.at[:, pl.ds(col0, CL)], m_v)

        def ref_copy(g, k):
            return pltpu.make_async_copy(
                ref_hbm.at[pl.ds(g * N, N), pl.ds(col0, CL)],
                r_v.at[k], srs[k])

        ref_copy(0, 0).start()

        @pl.loop(0, G // 2)
        def _(t):
            for k in range(2):
                g = 2 * t + k
                gn = jnp.minimum(g + 1, G - 1)
                ref_copy(gn, 1 - k).start()
                ref_copy(g, k).wait()
                b = g // A

                @pl.loop(0, CL // _LANES, unroll=4)
                def _(j):
                    sl = pl.ds(j * _LANES, _LANES)
                    mv = m_v[b, sl]
                    best = mv * r_v[k, 0, sl]
                    besti = jnp.zeros((_LANES,), jnp.int32)
                    for n in range(1, N):
                        p = mv * r_v[k, n, sl]
                        gt = p > best
                        best = jnp.where(gt, p, best)
                        besti = jnp.where(gt, jnp.int32(n), besti)
                    omax_v[g, sl] = best
                    oidx_v[g, sl] = besti

        ref_copy(G - 1, 0).wait()
        pltpu.sync_copy(omax_v, omax_hbm.at[:, pl.ds(col0, CL)])
        pltpu.sync_copy(oidx_v, oidx_hbm.at[:, pl.ds(col0, CL)])

    return sc_call


def kernel(input_mixed, ref_panel):
    B, A, N, L = ref_panel.shape
    sc_call = _make_sc_call(B, A, N, L)
    ref2d = ref_panel.reshape(B * A * N, L)
    omax, oidx = sc_call(input_mixed, ref2d)
    return omax.reshape(B, A, L), oidx.reshape(B, A, L)


# speedup vs baseline: 1.4816x; 1.0595x over previous
import functools

import jax
import jax.numpy as jnp
from jax import lax
from jax.experimental import pallas as pl
from jax.experimental.pallas import tpu as pltpu
from jax.experimental.pallas import tpu_sc as plsc

_NC, _NS, _LANES = 2, 16, 16
_NW = _NC * _NS


def _make_sc_call(B, A, N, L):
    G = B * A
    CL = L // _NW
    assert L % _NW == 0 and CL % _LANES == 0

    mesh = plsc.VectorSubcoreMesh(
        core_axis_name="c", subcore_axis_name="s",
        num_cores=_NC, num_subcores=_NS)

    @functools.partial(
        pl.kernel,
        out_type=(
            jax.ShapeDtypeStruct((G, L), jnp.float32),
            jax.ShapeDtypeStruct((G, L), jnp.int32),
        ),
        mesh=mesh,
        scratch_types=[
            pltpu.VMEM((B, CL), jnp.float32),
            pltpu.VMEM((2, N, CL), jnp.float32),
            pltpu.VMEM((G, CL), jnp.float32),
            pltpu.VMEM((G, CL), jnp.int32),
            pltpu.SemaphoreType.DMA,
            pltpu.SemaphoreType.DMA,
        ],
    )
    def sc_call(mixed_hbm, ref_hbm, omax_hbm, oidx_hbm, m_v, r_v, omax_v,
                oidx_v, sr0, sr1):
        srs = (sr0, sr1)
        wid = lax.axis_index("s") * _NC + lax.axis_index("c")
        col0 = wid * CL
        pltpu.sync_copy(mixed_hbm.at[:, pl.ds(col0, CL)], m_v)

        def ref_copy(g, k):
            return pltpu.make_async_copy(
                ref_hbm.at[pl.ds(g * N, N), pl.ds(col0, CL)],
                r_v.at[k], srs[k])

        ref_copy(0, 0).start()

        @pl.loop(0, G // 2)
        def _(t):
            for k in range(2):
                g = 2 * t + k
                gn = jnp.minimum(g + 1, G - 1)
                ref_copy(gn, 1 - k).start()
                ref_copy(g, k).wait()
                b = g // A

                @pl.loop(0, CL // _LANES, unroll=8)
                def _(j):
                    sl = pl.ds(j * _LANES, _LANES)
                    mv = m_v[b, sl]
                    best = mv * r_v[k, 0, sl]
                    besti = jnp.zeros((_LANES,), jnp.int32)
                    for n in range(1, N):
                        p = mv * r_v[k, n, sl]
                        gt = p > best
                        besti = jnp.where(gt, jnp.int32(n), besti)
                        best = jnp.maximum(best, p)
                    omax_v[g, sl] = best
                    oidx_v[g, sl] = besti

        ref_copy(G - 1, 0).wait()
        pltpu.sync_copy(omax_v, omax_hbm.at[:, pl.ds(col0, CL)])
        pltpu.sync_copy(oidx_v, oidx_hbm.at[:, pl.ds(col0, CL)])

    return sc_call


def kernel(input_mixed, ref_panel):
    B, A, N, L = ref_panel.shape
    sc_call = _make_sc_call(B, A, N, L)
    ref2d = ref_panel.reshape(B * A * N, L)
    omax, oidx = sc_call(input_mixed, ref2d)
    return omax.reshape(B, A, L), oidx.reshape(B, A, L)


# speedup vs baseline: 1.5140x; 1.0219x over previous
import functools

import jax
import jax.numpy as jnp
from jax import lax
from jax.experimental import pallas as pl
from jax.experimental.pallas import tpu as pltpu
from jax.experimental.pallas import tpu_sc as plsc

_NC, _NS, _LANES = 2, 16, 16
_NW = _NC * _NS


def _make_sc_call(B, A, N, L):
    G = B * A
    CL = L // _NW
    assert L % _NW == 0 and CL % _LANES == 0

    mesh = plsc.VectorSubcoreMesh(
        core_axis_name="c", subcore_axis_name="s",
        num_cores=_NC, num_subcores=_NS)

    @functools.partial(
        pl.kernel,
        out_type=(
            jax.ShapeDtypeStruct((B, A, L), jnp.float32),
            jax.ShapeDtypeStruct((B, A, L), jnp.int32),
        ),
        mesh=mesh,
        scratch_types=[
            pltpu.VMEM((B, CL), jnp.float32),
            pltpu.VMEM((2, N, CL), jnp.float32),
            pltpu.VMEM((B, A, CL), jnp.float32),
            pltpu.VMEM((B, A, CL), jnp.int32),
            pltpu.SemaphoreType.DMA,
            pltpu.SemaphoreType.DMA,
        ],
    )
    def sc_call(mixed_hbm, ref_hbm, omax_hbm, oidx_hbm, m_v, r_v, omax_v,
                oidx_v, sr0, sr1):
        srs = (sr0, sr1)
        wid = lax.axis_index("s") * _NC + lax.axis_index("c")
        col0 = wid * CL
        pltpu.sync_copy(mixed_hbm.at[:, pl.ds(col0, CL)], m_v)

        def ref_copy(g, k):
            return pltpu.make_async_copy(
                ref_hbm.at[pl.ds(g * N, N), pl.ds(col0, CL)],
                r_v.at[k], srs[k])

        ref_copy(0, 0).start()

        @pl.loop(0, G // 2)
        def _(t):
            for k in range(2):
                g = 2 * t + k
                gn = jnp.minimum(g + 1, G - 1)
                ref_copy(gn, 1 - k).start()
                ref_copy(g, k).wait()
                b = g // A
                a = g - b * A

                @pl.loop(0, CL // _LANES, unroll=8)
                def _(j):
                    sl = pl.ds(j * _LANES, _LANES)
                    mv = m_v[b, sl]
                    best = mv * r_v[k, 0, sl]
                    besti = jnp.zeros((_LANES,), jnp.int32)
                    for n in range(1, N):
                        p = mv * r_v[k, n, sl]
                        gt = p > best
                        besti = jnp.where(gt, jnp.int32(n), besti)
                        best = jnp.maximum(best, p)
                    omax_v[b, a, sl] = best
                    oidx_v[b, a, sl] = besti

        ref_copy(G - 1, 0).wait()
        h1 = pltpu.async_copy(omax_v, omax_hbm.at[:, :, pl.ds(col0, CL)], sr0)
        h2 = pltpu.async_copy(oidx_v, oidx_hbm.at[:, :, pl.ds(col0, CL)], sr1)
        h1.wait()
        h2.wait()

    return sc_call


def kernel(input_mixed, ref_panel):
    B, A, N, L = ref_panel.shape
    sc_call = _make_sc_call(B, A, N, L)
    ref2d = ref_panel.reshape(B * A * N, L)
    return sc_call(input_mixed, ref2d)
